# Initial kernel scaffold; baseline (speedup 1.0000x reference)
import jax
import jax.numpy as jnp
from jax.experimental import pallas as pl


def kernel(src_type_idx, edge_type_idx, dst_type_idx, node_table, edge_table):
    raise NotImplementedError("write your pallas kernel here")


# speedup vs baseline: 3.2691x; 3.2691x over previous
import functools

import jax
import jax.numpy as jnp
from jax import lax
from jax.experimental import pallas as pl
from jax.experimental.pallas import tpu as pltpu
from jax.experimental.pallas import tpu_sc as plsc

NUM_NODE_TYPES = 1000
NUM_EDGE_TYPES = 1000
EMBED_DIM = 64
BATCH = 16384

NC = 2
NS = 16
NW = NC * NS
B_PER_W = BATCH // NW
CHUNK = 128
NCHUNK = B_PER_W // CHUNK
LANES = 16
VECS_PER_ROW = EMBED_DIM // LANES


def _sc_body(node_hbm, edge_hbm, src_hbm, edg_hbm, dst_hbm, out_hbm,
             src_idx, edg_idx, dst_idx, a_rows, b_rows, c_rows, sem):
    wid = lax.axis_index("s") * NC + lax.axis_index("c")
    base = wid * B_PER_W

    pltpu.sync_copy(src_hbm.at[wid], src_idx)
    pltpu.sync_copy(edg_hbm.at[wid], edg_idx)
    pltpu.sync_copy(dst_hbm.at[wid], dst_idx)

    copies = []
    for j in range(NCHUNK):
        rows = pl.ds(j * CHUNK, CHUNK)
        copies.append(pltpu.async_copy(node_hbm.at[src_idx.at[j]],
                                       a_rows.at[rows], sem))
        copies.append(pltpu.async_copy(edge_hbm.at[edg_idx.at[j]],
                                       b_rows.at[rows], sem))
        copies.append(pltpu.async_copy(node_hbm.at[dst_idx.at[j]],
                                       c_rows.at[rows], sem))
    for c in copies:
        c.wait()

    def row_body(r, carry):
        for v in range(VECS_PER_ROW):
            cols = pl.ds(v * LANES, LANES)
            a_rows[r, cols] = a_rows[r, cols] - b_rows[r, cols] + c_rows[r, cols]
        return carry

    lax.fori_loop(0, B_PER_W, row_body, 0, unroll=4)

    pltpu.sync_copy(a_rows, out_hbm.at[pl.ds(base, B_PER_W)])


@jax.jit
def _sc_call(node_table, edge_table, src3, edg3, dst3):
    mesh = plsc.VectorSubcoreMesh(core_axis_name="c", subcore_axis_name="s")
    return pl.kernel(
        _sc_body,
        mesh=mesh,
        compiler_params=pltpu.CompilerParams(use_tc_tiling_on_sc=False),
        out_type=jax.ShapeDtypeStruct((BATCH, EMBED_DIM), jnp.float32),
        scratch_types=[
            pltpu.VMEM((NCHUNK, CHUNK), jnp.int32),
            pltpu.VMEM((NCHUNK, CHUNK), jnp.int32),
            pltpu.VMEM((NCHUNK, CHUNK), jnp.int32),
            pltpu.VMEM((B_PER_W, EMBED_DIM), jnp.float32),
            pltpu.VMEM((B_PER_W, EMBED_DIM), jnp.float32),
            pltpu.VMEM((B_PER_W, EMBED_DIM), jnp.float32),
            pltpu.SemaphoreType.DMA,
        ],
    )(node_table, edge_table, src3, edg3, dst3)


def kernel(src_type_idx, edge_type_idx, dst_type_idx, node_table, edge_table):
    src3 = src_type_idx.astype(jnp.int32).reshape(NW, NCHUNK, CHUNK)
    edg3 = edge_type_idx.astype(jnp.int32).reshape(NW, NCHUNK, CHUNK)
    dst3 = dst_type_idx.astype(jnp.int32).reshape(NW, NCHUNK, CHUNK)
    return _sc_call(node_table, edge_table, src3, edg3, dst3)


# speedup vs baseline: 4.1015x; 1.2546x over previous
import functools

import jax
import jax.numpy as jnp
from jax import lax
from jax.experimental import pallas as pl
from jax.experimental.pallas import tpu as pltpu
from jax.experimental.pallas import tpu_sc as plsc

NUM_NODE_TYPES = 1000
NUM_EDGE_TYPES = 1000
EMBED_DIM = 64
BATCH = 16384

NC = 2
NS = 16
NW = NC * NS
B_PER_W = BATCH // NW
CHUNK = 128
NCHUNK = B_PER_W // CHUNK
LANES = 16
VECS_PER_ROW = EMBED_DIM // LANES
TAB_ROWS = NUM_NODE_TYPES + NUM_EDGE_TYPES
STAGE_ROWS = TAB_ROWS // NS


def _sc_body(node_hbm, edge_hbm, src_hbm, edg_hbm, dst_hbm, out_hbm,
             tab_sh, src_idx, edg_idx, dst_idx, a_rows, b_rows, c_rows,
             gsems, osem):
    cid = lax.axis_index("c")
    sid = lax.axis_index("s")
    wid = sid * NC + cid
    base = wid * B_PER_W

    pltpu.sync_copy(src_hbm.at[wid], src_idx)
    pltpu.sync_copy(edg_hbm.at[wid], edg_idx)
    pltpu.sync_copy(dst_hbm.at[wid], dst_idx)

    row0 = sid * STAGE_ROWS

    @pl.when(sid < NS // 2)
    def _():
        pltpu.sync_copy(node_hbm.at[pl.ds(row0, STAGE_ROWS)],
                        tab_sh.at[pl.ds(row0, STAGE_ROWS)])

    @pl.when(sid >= NS // 2)
    def _():
        pltpu.sync_copy(edge_hbm.at[pl.ds(row0 - NUM_NODE_TYPES, STAGE_ROWS)],
                        tab_sh.at[pl.ds(row0, STAGE_ROWS)])

    plsc.subcore_barrier()

    gather_waits = []
    for j in range(NCHUNK):
        rows = pl.ds(j * CHUNK, CHUNK)
        sem = gsems.at[j]
        gather_waits.append((
            pltpu.async_copy(tab_sh.at[src_idx.at[j]], a_rows.at[rows], sem),
            pltpu.async_copy(tab_sh.at[edg_idx.at[j]], b_rows.at[rows], sem),
            pltpu.async_copy(tab_sh.at[dst_idx.at[j]], c_rows.at[rows], sem),
        ))

    out_waits = []
    for j in range(NCHUNK):
        for w in gather_waits[j]:
            w.wait()

        @plsc.parallel_loop(j * CHUNK * VECS_PER_ROW,
                            (j + 1) * CHUNK * VECS_PER_ROW, unroll=4)
        def _vec(i):
            r = i // VECS_PER_ROW
            cols = pl.ds((i % VECS_PER_ROW) * LANES, LANES)
            a_rows[r, cols] = a_rows[r, cols] - b_rows[r, cols] + c_rows[r, cols]

        out_waits.append(pltpu.async_copy(
            a_rows.at[pl.ds(j * CHUNK, CHUNK)],
            out_hbm.at[pl.ds(base + j * CHUNK, CHUNK)], osem))

    for w in out_waits:
        w.wait()


@jax.jit
def _sc_call(node_table, edge_table, src3, edg3, dst3):
    mesh = plsc.VectorSubcoreMesh(core_axis_name="c", subcore_axis_name="s")
    return pl.kernel(
        _sc_body,
        mesh=mesh,
        compiler_params=pltpu.CompilerParams(use_tc_tiling_on_sc=False),
        out_type=jax.ShapeDtypeStruct((BATCH, EMBED_DIM), jnp.float32),
        scratch_types=[
            pltpu.VMEM_SHARED((TAB_ROWS, EMBED_DIM), jnp.float32),
            pltpu.VMEM((NCHUNK, CHUNK), jnp.int32),
            pltpu.VMEM((NCHUNK, CHUNK), jnp.int32),
            pltpu.VMEM((NCHUNK, CHUNK), jnp.int32),
            pltpu.VMEM((B_PER_W, EMBED_DIM), jnp.float32),
            pltpu.VMEM((B_PER_W, EMBED_DIM), jnp.float32),
            pltpu.VMEM((B_PER_W, EMBED_DIM), jnp.float32),
            pltpu.SemaphoreType.DMA((NCHUNK,)),
            pltpu.SemaphoreType.DMA,
        ],
    )(node_table, edge_table, src3, edg3, dst3)


def kernel(src_type_idx, edge_type_idx, dst_type_idx, node_table, edge_table):
    src3 = src_type_idx.astype(jnp.int32).reshape(NW, NCHUNK, CHUNK)
    edg3 = (edge_type_idx.astype(jnp.int32) + NUM_NODE_TYPES).reshape(
        NW, NCHUNK, CHUNK)
    dst3 = dst_type_idx.astype(jnp.int32).reshape(NW, NCHUNK, CHUNK)
    return _sc_call(node_table, edge_table, src3, edg3, dst3)
